# baseline (speedup 1.0000x reference)
import functools

import jax
import jax.numpy as jnp
from jax import lax
from jax.experimental import pallas as pl
from jax.experimental.pallas import tpu as pltpu
from jax.experimental.pallas import tpu_sc as plsc

A = 16
K = 8
F = 26
V = 1000
D = 32
RAW = 128
B = 1024
H1 = 64
H2 = 64
C_BW = 1000.0
FD = F * D

NC = 2
NS = 16
BH = B // 2
NG = BH // 16


@functools.cache
def _make_sc_gather():
    mesh = plsc.VectorSubcoreMesh(
        core_axis_name="c", subcore_axis_name="s",
        num_cores=NC, num_subcores=NS)

    @functools.partial(
        pl.kernel,
        out_type=jax.ShapeDtypeStruct((A * FD, B), jnp.float32),
        mesh=mesh,
        scratch_types=[
            pltpu.VMEM((D, V), jnp.float32),
            pltpu.VMEM((D, BH), jnp.float32),
            pltpu.VMEM((BH,), jnp.int32),
        ],
        compiler_params=pltpu.CompilerParams(
            use_tc_tiling_on_sc=True, needs_layout_passes=False),
    )
    def sc_gather(embt_hbm, xit_hbm, out_hbm, slice_v, st_v, xi_v):
        wid = lax.axis_index("s") * NC + lax.axis_index("c")
        a = wid // 2
        b0 = (wid % 2) * BH

        def per_f(f, _):
            pltpu.sync_copy(embt_hbm.at[a, f], slice_v)
            pltpu.sync_copy(xit_hbm.at[pl.ds(f * B + b0, BH)], xi_v)

            def per_g(g, carry):
                v16 = xi_v[pl.ds(g * 16, 16)]
                for d in range(D):
                    dvec = jnp.full((16,), d, jnp.int32)
                    vals = plsc.load_gather(slice_v, [dvec, v16])
                    st_v[d, pl.ds(g * 16, 16)] = vals
                return carry

            lax.fori_loop(0, NG, per_g, 0)
            pltpu.sync_copy(
                st_v,
                out_hbm.at[pl.ds(a * FD + f * D, D), pl.ds(b0, BH)])
            return 0

        lax.fori_loop(0, F, per_f, 0)

    return sc_gather


def _dot0(x, y, prec=lax.Precision.DEFAULT):
    return lax.dot_general(x, y, (((0,), (0,)), ((), ())),
                           preferred_element_type=jnp.float32,
                           precision=prec)


def _tc_body(emb_ref, xv_ref, x_ref, ap_ref, bias_ref,
             w1_ref, b1_ref, w2_ref, b2_ref,
             out_ref, es_ref, xe_ref, wsel_ref, wb_ref):
    a = pl.program_id(0)

    @pl.when(a == 0)
    def _():
        xt = x_ref[...]
        ap = ap_ref[...]
        x2 = jnp.sum(xt * xt, axis=0, keepdims=True)
        xa = lax.dot_general(ap, xt, (((1,), (0,)), ((), ())),
                             preferred_element_type=jnp.float32,
                             precision=lax.Precision.HIGHEST)
        a2 = jnp.sum(ap * ap, axis=1, keepdims=True)
        dist = a2 - 2.0 * xa + x2
        sim = jnp.exp(dist * (-1.0 / C_BW))
        iota0 = lax.broadcasted_iota(jnp.int32, (A, B), 0)
        work = sim
        sel = jnp.zeros((A, B), jnp.float32)
        for _ in range(K):
            m = jnp.max(work, axis=0, keepdims=True)
            cand = jnp.where(work >= m, iota0, A + 1)
            amin = jnp.min(cand, axis=0, keepdims=True)
            pick = iota0 == amin
            sel = jnp.where(pick, 1.0, sel)
            work = jnp.where(pick, -1e30, work)
        wun = sim * sel
        den = jnp.sum(wun, axis=0, keepdims=True) + 1e-12
        wsel = wun / den
        wsel_ref[...] = wsel
        wb_ref[...] = _dot0(bias_ref[...], wsel)
        rmat = (lax.broadcasted_iota(jnp.int32, (FD, F), 0) // D
                == lax.broadcasted_iota(jnp.int32, (FD, F), 1)
                ).astype(jnp.float32)
        xe_ref[...] = lax.dot_general(
            rmat, xv_ref[...], (((1,), (0,)), ((), ())),
            preferred_element_type=jnp.float32)

    es = emb_ref[0] * xe_ref[...]
    es_ref[...] = es
    smat = (lax.broadcasted_iota(jnp.int32, (FD, D), 0) % D
            == lax.broadcasted_iota(jnp.int32, (FD, D), 1)).astype(jnp.float32)
    cat = jnp.concatenate([w1_ref[a], smat], axis=1)
    hs = _dot0(cat, es_ref[...])
    s = hs[H1:, :]
    fm = 0.5 * (jnp.sum(s * s, axis=0, keepdims=True)
                - jnp.sum(es * es, axis=0, keepdims=True))

    h1 = jnp.maximum(hs[:H1, :] + b1_ref[0], 0.0)
    h2 = jnp.maximum(_dot0(w2_ref[a], h1) + b2_ref[0], 0.0)
    deep = jnp.sum(h2, axis=0, keepdims=True)

    amask = lax.broadcasted_iota(jnp.int32, (A, B), 0) == a
    wrow = jnp.sum(jnp.where(amask, wsel_ref[...], 0.0),
                   axis=0, keepdims=True)
    contrib = wrow * (fm + deep)

    @pl.when(a == 0)
    def _():
        out_ref[...] = wb_ref[...] + contrib

    @pl.when(a != 0)
    def _():
        out_ref[...] = out_ref[...] + contrib


_tc_fused = pl.pallas_call(
    _tc_body,
    grid=(A,),
    in_specs=[
        pl.BlockSpec((1, FD, B), lambda a: (a, 0, 0)),
        pl.BlockSpec((F, B), lambda a: (0, 0)),
        pl.BlockSpec((RAW, B), lambda a: (0, 0)),
        pl.BlockSpec((A, RAW), lambda a: (0, 0)),
        pl.BlockSpec((A, 1), lambda a: (0, 0)),
        pl.BlockSpec((A, FD, H1), lambda a: (0, 0, 0)),
        pl.BlockSpec((1, H1, 1), lambda a: (a, 0, 0)),
        pl.BlockSpec((A, H1, H2), lambda a: (0, 0, 0)),
        pl.BlockSpec((1, H2, 1), lambda a: (a, 0, 0)),
    ],
    out_specs=pl.BlockSpec((1, B), lambda a: (0, 0)),
    out_shape=jax.ShapeDtypeStruct((1, B), jnp.float32),
    scratch_shapes=[
        pltpu.VMEM((FD, B), jnp.float32),
        pltpu.VMEM((FD, B), jnp.float32),
        pltpu.VMEM((A, B), jnp.float32),
        pltpu.VMEM((1, B), jnp.float32),
    ],
    compiler_params=pltpu.CompilerParams(
        dimension_semantics=("arbitrary",)),
)


def kernel(Xi, Xv, X, anchor_points, bias, emb_tables, W1, b1, W2, b2):
    embt = jnp.transpose(emb_tables, (0, 1, 3, 2))
    xit = jnp.transpose(Xi).astype(jnp.int32).reshape(F * B)
    rows = _make_sc_gather()(embt, xit)
    emb3 = rows.reshape(A, FD, B)
    out = _tc_fused(emb3, jnp.transpose(Xv), jnp.transpose(X),
                    anchor_points, bias, W1,
                    b1.reshape(A, H1, 1), W2, b2.reshape(A, H2, 1))
    return out.reshape(B)


# speedup vs baseline: 1.4528x; 1.4528x over previous
import functools

import jax
import jax.numpy as jnp
from jax import lax
from jax.experimental import pallas as pl
from jax.experimental.pallas import tpu as pltpu
from jax.experimental.pallas import tpu_sc as plsc

A = 16
K = 8
F = 26
V = 1000
D = 32
RAW = 128
B = 1024
H1 = 64
H2 = 64
C_BW = 1000.0
FD = F * D

NC = 2
NS = 16
BH = B // 2
NG = BH // 16


@functools.cache
def _make_sc_gather():
    mesh = plsc.VectorSubcoreMesh(
        core_axis_name="c", subcore_axis_name="s",
        num_cores=NC, num_subcores=NS)

    @functools.partial(
        pl.kernel,
        out_type=jax.ShapeDtypeStruct((A * FD, B), jnp.float32),
        mesh=mesh,
        scratch_types=[
            pltpu.VMEM((D, V), jnp.float32),
            pltpu.VMEM((D, V), jnp.float32),
            pltpu.VMEM((D, BH), jnp.float32),
            pltpu.VMEM((D, BH), jnp.float32),
            pltpu.VMEM((F * BH,), jnp.int32),
            pltpu.SemaphoreType.DMA,
            pltpu.SemaphoreType.DMA,
            pltpu.SemaphoreType.DMA,
            pltpu.SemaphoreType.DMA,
            pltpu.SemaphoreType.DMA,
        ],
        compiler_params=pltpu.CompilerParams(
            use_tc_tiling_on_sc=True, needs_layout_passes=False),
    )
    def sc_gather(embt_hbm, xit_hbm, out_hbm,
                  sl0, sl1, st0, st1, xi_v, sa0, sa1, sb0, sb1, sx):
        wid = lax.axis_index("s") * NC + lax.axis_index("c")
        a = wid // 2
        b0 = (wid % 2) * BH
        sls = (sl0, sl1)
        sts = (st0, st1)
        sas = (sa0, sa1)
        sbs = (sb0, sb1)

        for f in range(F):
            pltpu.async_copy(xit_hbm.at[pl.ds(f * B + b0, BH)],
                             xi_v.at[pl.ds(f * BH, BH)], sx)
        for f in range(F):
            pltpu.make_async_copy(xit_hbm.at[pl.ds(f * B + b0, BH)],
                                  xi_v.at[pl.ds(f * BH, BH)], sx).wait()

        def out_dst(f):
            return out_hbm.at[pl.ds(a * FD + f * D, D), pl.ds(b0, BH)]

        pltpu.async_copy(embt_hbm.at[a, 0], sl0, sa0)

        def pair(p, _):
            for par in range(2):
                f = 2 * p + par
                pltpu.make_async_copy(
                    embt_hbm.at[a, f], sls[par], sas[par]).wait()

                @pl.when(f + 1 < F)
                def _():
                    pltpu.async_copy(
                        embt_hbm.at[a, f + 1], sls[1 - par], sas[1 - par])

                @pl.when(f >= 2)
                def _():
                    pltpu.make_async_copy(
                        sts[par], out_dst(f - 2), sbs[par]).wait()

                def per_g(g, carry):
                    v16 = xi_v[pl.ds(f * BH + g * 16, 16)]
                    for d in range(D):
                        dvec = jnp.full((16,), d, jnp.int32)
                        vals = plsc.load_gather(sls[par], [dvec, v16])
                        sts[par][d, pl.ds(g * 16, 16)] = vals
                    return carry

                lax.fori_loop(0, NG, per_g, 0)
                pltpu.async_copy(sts[par], out_dst(f), sbs[par])
            return 0

        lax.fori_loop(0, F // 2, pair, 0)
        pltpu.make_async_copy(st0, out_dst(F - 2), sb0).wait()
        pltpu.make_async_copy(st1, out_dst(F - 1), sb1).wait()

    return sc_gather


def _dot0(x, y, prec=lax.Precision.DEFAULT):
    return lax.dot_general(x, y, (((0,), (0,)), ((), ())),
                           preferred_element_type=jnp.float32,
                           precision=prec)


def _tc_body(emb_ref, xv_ref, x_ref, ap_ref, bias_ref,
             w1_ref, b1_ref, w2_ref, b2_ref,
             out_ref, es_ref, xe_ref, wsel_ref, wb_ref):
    a = pl.program_id(0)

    @pl.when(a == 0)
    def _():
        xt = x_ref[...]
        ap = ap_ref[...]
        x2 = jnp.sum(xt * xt, axis=0, keepdims=True)
        xa = lax.dot_general(ap, xt, (((1,), (0,)), ((), ())),
                             preferred_element_type=jnp.float32,
                             precision=lax.Precision.HIGHEST)
        a2 = jnp.sum(ap * ap, axis=1, keepdims=True)
        dist = a2 - 2.0 * xa + x2
        sim = jnp.exp(dist * (-1.0 / C_BW))
        iota0 = lax.broadcasted_iota(jnp.int32, (A, B), 0)
        work = sim
        sel = jnp.zeros((A, B), jnp.float32)
        for _ in range(K):
            m = jnp.max(work, axis=0, keepdims=True)
            cand = jnp.where(work >= m, iota0, A + 1)
            amin = jnp.min(cand, axis=0, keepdims=True)
            pick = iota0 == amin
            sel = jnp.where(pick, 1.0, sel)
            work = jnp.where(pick, -1e30, work)
        wun = sim * sel
        den = jnp.sum(wun, axis=0, keepdims=True) + 1e-12
        wsel = wun / den
        wsel_ref[...] = wsel
        wb_ref[...] = _dot0(bias_ref[...], wsel)
        rmat = (lax.broadcasted_iota(jnp.int32, (FD, F), 0) // D
                == lax.broadcasted_iota(jnp.int32, (FD, F), 1)
                ).astype(jnp.float32)
        xe_ref[...] = lax.dot_general(
            rmat, xv_ref[...], (((1,), (0,)), ((), ())),
            preferred_element_type=jnp.float32)

    es = emb_ref[0] * xe_ref[...]
    es_ref[...] = es
    smat = (lax.broadcasted_iota(jnp.int32, (FD, D), 0) % D
            == lax.broadcasted_iota(jnp.int32, (FD, D), 1)).astype(jnp.float32)
    cat = jnp.concatenate([w1_ref[a], smat], axis=1)
    hs = _dot0(cat, es_ref[...])
    s = hs[H1:, :]
    fm = 0.5 * (jnp.sum(s * s, axis=0, keepdims=True)
                - jnp.sum(es * es, axis=0, keepdims=True))

    h1 = jnp.maximum(hs[:H1, :] + b1_ref[0], 0.0)
    h2 = jnp.maximum(_dot0(w2_ref[a], h1) + b2_ref[0], 0.0)
    deep = jnp.sum(h2, axis=0, keepdims=True)

    amask = lax.broadcasted_iota(jnp.int32, (A, B), 0) == a
    wrow = jnp.sum(jnp.where(amask, wsel_ref[...], 0.0),
                   axis=0, keepdims=True)
    contrib = wrow * (fm + deep)

    @pl.when(a == 0)
    def _():
        out_ref[...] = wb_ref[...] + contrib

    @pl.when(a != 0)
    def _():
        out_ref[...] = out_ref[...] + contrib


_tc_fused = pl.pallas_call(
    _tc_body,
    grid=(A,),
    in_specs=[
        pl.BlockSpec((1, FD, B), lambda a: (a, 0, 0)),
        pl.BlockSpec((F, B), lambda a: (0, 0)),
        pl.BlockSpec((RAW, B), lambda a: (0, 0)),
        pl.BlockSpec((A, RAW), lambda a: (0, 0)),
        pl.BlockSpec((A, 1), lambda a: (0, 0)),
        pl.BlockSpec((A, FD, H1), lambda a: (0, 0, 0)),
        pl.BlockSpec((1, H1, 1), lambda a: (a, 0, 0)),
        pl.BlockSpec((A, H1, H2), lambda a: (0, 0, 0)),
        pl.BlockSpec((1, H2, 1), lambda a: (a, 0, 0)),
    ],
    out_specs=pl.BlockSpec((1, B), lambda a: (0, 0)),
    out_shape=jax.ShapeDtypeStruct((1, B), jnp.float32),
    scratch_shapes=[
        pltpu.VMEM((FD, B), jnp.float32),
        pltpu.VMEM((FD, B), jnp.float32),
        pltpu.VMEM((A, B), jnp.float32),
        pltpu.VMEM((1, B), jnp.float32),
    ],
    compiler_params=pltpu.CompilerParams(
        dimension_semantics=("arbitrary",)),
)


def kernel(Xi, Xv, X, anchor_points, bias, emb_tables, W1, b1, W2, b2):
    embt = jnp.transpose(emb_tables, (0, 1, 3, 2))
    xit = jnp.transpose(Xi).astype(jnp.int32).reshape(F * B)
    rows = _make_sc_gather()(embt, xit)
    emb3 = rows.reshape(A, FD, B)
    out = _tc_fused(emb3, jnp.transpose(Xv), jnp.transpose(X),
                    anchor_points, bias, W1,
                    b1.reshape(A, H1, 1), W2, b2.reshape(A, H2, 1))
    return out.reshape(B)
